# Initial kernel scaffold; baseline (speedup 1.0000x reference)
import jax
import jax.numpy as jnp
from jax.experimental import pallas as pl


def kernel(x, edge_index, edge_weight, W1, b1, W2, b2, Ws, bs):
    raise NotImplementedError("write your pallas kernel here")


# speedup vs baseline: 1.9328x; 1.9328x over previous
import functools

import jax
import jax.numpy as jnp
from jax import lax
from jax.experimental import pallas as pl
from jax.experimental.pallas import tpu as pltpu

NN = 10000
EE = 320000
FF = 128
HH = 128
KK = 64
BLK = 2000
GRID = NN // BLK
HI = lax.Precision.DEFAULT
NP_ = 10240


def _selu(x):
    alpha = 1.6732632423543772848170429916717
    scale = 1.0507009873554804934193349852946
    return scale * jnp.where(x > 0, x, alpha * (jnp.exp(jnp.minimum(x, 0.0)) - 1.0))


def _k1_body(degp_ref, x_ref, w1_ref, h1_ref, u1_ref, dinv_ref):
    deg = degp_ref[0] + degp_ref[1] + 1.0
    dinv = lax.rsqrt(deg)
    h1 = jnp.dot(x_ref[...], w1_ref[...], precision=HI,
                 preferred_element_type=jnp.float32)
    h1_ref[...] = h1
    u1_ref[...] = h1 * dinv
    dinv_ref[...] = dinv


def _k1(degp3, x, W1):
    return pl.pallas_call(
        _k1_body,
        grid=(GRID,),
        in_specs=[
            pl.BlockSpec((2, BLK, 1), lambda i: (0, i, 0)),
            pl.BlockSpec((BLK, FF), lambda i: (i, 0)),
            pl.BlockSpec((FF, HH), lambda i: (0, 0)),
        ],
        out_specs=[
            pl.BlockSpec((BLK, HH), lambda i: (i, 0)),
            pl.BlockSpec((BLK, HH), lambda i: (i, 0)),
            pl.BlockSpec((BLK, 1), lambda i: (i, 0)),
        ],
        out_shape=[
            jax.ShapeDtypeStruct((NN, HH), jnp.float32),
            jax.ShapeDtypeStruct((NN, HH), jnp.float32),
            jax.ShapeDtypeStruct((NN, 1), jnp.float32),
        ],
    )(degp3, x, W1)


def _k2_body(sp_ref, h1_ref, dinv_ref, b1_ref, w2_ref, h2_ref, u2_ref):
    dinv = dinv_ref[...]
    agg = dinv * (sp_ref[0] + sp_ref[1]) + (dinv * dinv) * h1_ref[...]
    hsel = _selu(agg + b1_ref[...])
    h2 = jnp.dot(hsel, w2_ref[...], precision=HI,
                 preferred_element_type=jnp.float32)
    h2_ref[...] = h2
    u2_ref[...] = h2 * dinv


def _k2(sp3, h1, dinv, b1, W2):
    return pl.pallas_call(
        _k2_body,
        grid=(GRID,),
        in_specs=[
            pl.BlockSpec((2, BLK, HH), lambda i: (0, i, 0)),
            pl.BlockSpec((BLK, HH), lambda i: (i, 0)),
            pl.BlockSpec((BLK, 1), lambda i: (i, 0)),
            pl.BlockSpec((1, HH), lambda i: (0, 0)),
            pl.BlockSpec((HH, HH), lambda i: (0, 0)),
        ],
        out_specs=[
            pl.BlockSpec((BLK, HH), lambda i: (i, 0)),
            pl.BlockSpec((BLK, HH), lambda i: (i, 0)),
        ],
        out_shape=[
            jax.ShapeDtypeStruct((NN, HH), jnp.float32),
            jax.ShapeDtypeStruct((NN, HH), jnp.float32),
        ],
    )(sp3, h1, dinv, b1, W2)


def _k3_body(sp_ref, h2_ref, dinv_ref, b2_ref, ws_ref, bs_ref, degs_ref,
             s_ref, sth_ref, ss_ref, ca_ref, csum_ref, sumdeg_ref):
    i = pl.program_id(0)

    @pl.when(i == 0)
    def _():
        sth_ref[...] = jnp.zeros_like(sth_ref)
        ss_ref[...] = jnp.zeros_like(ss_ref)
        ca_ref[...] = jnp.zeros_like(ca_ref)
        csum_ref[...] = jnp.zeros_like(csum_ref)
        sumdeg_ref[...] = jnp.zeros_like(sumdeg_ref)

    dinv = dinv_ref[...]
    agg = dinv * (sp_ref[0] + sp_ref[1]) + (dinv * dinv) * h2_ref[...]
    hf = _selu(agg + b2_ref[...])
    logits = jnp.dot(hf, ws_ref[...], precision=HI,
                     preferred_element_type=jnp.float32) + bs_ref[...]
    z = logits - jnp.max(logits, axis=-1, keepdims=True)
    ez = jnp.exp(z)
    sblk = ez / jnp.sum(ez, axis=-1, keepdims=True)
    s_ref[...] = sblk
    dot_t = functools.partial(lax.dot_general,
                              dimension_numbers=(((0,), (0,)), ((), ())),
                              precision=HI,
                              preferred_element_type=jnp.float32)
    sth_ref[...] += dot_t(sblk, hf)
    ss_ref[...] += dot_t(sblk, sblk)
    degs = degs_ref[...]
    ca_ref[...] += dot_t(degs, sblk)
    csum_ref[...] += jnp.sum(sblk, axis=0, keepdims=True)
    sumdeg_ref[...] += jnp.sum(degs, keepdims=True).reshape(1, 1)


def _k3(sp3, h2, dinv, b2, Ws, bs, degs):
    return pl.pallas_call(
        _k3_body,
        grid=(GRID,),
        in_specs=[
            pl.BlockSpec((2, BLK, HH), lambda i: (0, i, 0)),
            pl.BlockSpec((BLK, HH), lambda i: (i, 0)),
            pl.BlockSpec((BLK, 1), lambda i: (i, 0)),
            pl.BlockSpec((1, HH), lambda i: (0, 0)),
            pl.BlockSpec((HH, KK), lambda i: (0, 0)),
            pl.BlockSpec((1, KK), lambda i: (0, 0)),
            pl.BlockSpec((BLK, 1), lambda i: (i, 0)),
        ],
        out_specs=[
            pl.BlockSpec((BLK, KK), lambda i: (i, 0)),
            pl.BlockSpec((KK, HH), lambda i: (0, 0)),
            pl.BlockSpec((KK, KK), lambda i: (0, 0)),
            pl.BlockSpec((1, KK), lambda i: (0, 0)),
            pl.BlockSpec((1, KK), lambda i: (0, 0)),
            pl.BlockSpec((1, 1), lambda i: (0, 0)),
        ],
        out_shape=[
            jax.ShapeDtypeStruct((NN, KK), jnp.float32),
            jax.ShapeDtypeStruct((KK, HH), jnp.float32),
            jax.ShapeDtypeStruct((KK, KK), jnp.float32),
            jax.ShapeDtypeStruct((1, KK), jnp.float32),
            jax.ShapeDtypeStruct((1, KK), jnp.float32),
            jax.ShapeDtypeStruct((1, 1), jnp.float32),
        ],
    )(sp3, h2, dinv, b2, Ws, bs, degs)


def _k4_body(tp_ref, s_ref, sth_ref, ss_ref, ca_ref, csum_ref, sumdeg_ref,
             out_ref, oadj_ref, spec_ref, orth_ref, clus_ref):
    i = pl.program_id(0)

    @pl.when(i == 0)
    def _():
        oadj_ref[...] = jnp.zeros_like(oadj_ref)

    t = tp_ref[0] + tp_ref[1]
    oadj_ref[...] += lax.dot_general(
        t, s_ref[...], dimension_numbers=(((0,), (0,)), ((), ())),
        precision=HI, preferred_element_type=jnp.float32)

    @pl.when(i == GRID - 1)
    def _():
        out_ref[...] = _selu(sth_ref[...])
        oadj = oadj_ref[...]
        m = sumdeg_ref[0, 0] / 2.0
        eye = (lax.broadcasted_iota(jnp.int32, (KK, KK), 0) ==
               lax.broadcasted_iota(jnp.int32, (KK, KK), 1)).astype(jnp.float32)
        ca = ca_ref[...]
        tr_oadj = jnp.sum(oadj * eye)
        tr_norm = jnp.sum(ca * ca) / (2.0 * m)
        spec_ref[...] = (-(tr_oadj - tr_norm) / (2.0 * m)).reshape(1, 1)
        ss = ss_ref[...]
        nss = jnp.sqrt(jnp.sum(ss * ss))
        d = ss / nss - eye / 8.0
        orth_ref[...] = jnp.sqrt(jnp.sum(d * d)).reshape(1, 1)
        csum = csum_ref[...]
        clus_ref[...] = (jnp.sqrt(jnp.sum(csum * csum)) / NN * 8.0
                         - 1.0).reshape(1, 1)


def _k4(tp3, s, sth, ss, ca, csum, sumdeg):
    return pl.pallas_call(
        _k4_body,
        grid=(GRID,),
        in_specs=[
            pl.BlockSpec((2, BLK, KK), lambda i: (0, i, 0)),
            pl.BlockSpec((BLK, KK), lambda i: (i, 0)),
            pl.BlockSpec((KK, HH), lambda i: (0, 0)),
            pl.BlockSpec((KK, KK), lambda i: (0, 0)),
            pl.BlockSpec((1, KK), lambda i: (0, 0)),
            pl.BlockSpec((1, KK), lambda i: (0, 0)),
            pl.BlockSpec((1, 1), lambda i: (0, 0)),
        ],
        out_specs=[
            pl.BlockSpec((KK, HH), lambda i: (0, 0)),
            pl.BlockSpec((KK, KK), lambda i: (0, 0)),
            pl.BlockSpec((1, 1), lambda i: (0, 0)),
            pl.BlockSpec((1, 1), lambda i: (0, 0)),
            pl.BlockSpec((1, 1), lambda i: (0, 0)),
        ],
        out_shape=[
            jax.ShapeDtypeStruct((KK, HH), jnp.float32),
            jax.ShapeDtypeStruct((KK, KK), jnp.float32),
            jax.ShapeDtypeStruct((1, 1), jnp.float32),
            jax.ShapeDtypeStruct((1, 1), jnp.float32),
            jax.ShapeDtypeStruct((1, 1), jnp.float32),
        ],
    )(tp3, s, sth, ss, ca, csum, sumdeg)


def _seg_scalar(ew, src, dst):
    degd = jax.ops.segment_sum(ew, dst, num_segments=NN)
    degs = jax.ops.segment_sum(ew, src, num_segments=NN)
    z = jnp.zeros_like(degd)
    degp3 = jnp.stack([degd, z])[:, :, None]
    return degp3, degs[:, None]


def _seg_rows(u, ew, src, dst):
    agg = jax.ops.segment_sum(u[src] * ew[:, None], dst, num_segments=NN)
    return jnp.stack([agg, jnp.zeros_like(agg)])


def kernel(x, edge_index, edge_weight, W1, b1, W2, b2, Ws, bs):
    src = edge_index[0]
    dst = edge_index[1]
    ew = edge_weight
    b1r = b1.reshape(1, HH)
    b2r = b2.reshape(1, HH)
    bsr = bs.reshape(1, KK)

    degp3, degs = _seg_scalar(ew, src, dst)
    h1, u1, dinv = _k1(degp3, x, W1)
    s1 = _seg_rows(u1, ew, src, dst)
    h2, u2 = _k2(s1, h1, dinv, b1r, W2)
    s2 = _seg_rows(u2, ew, src, dst)
    s, sth, ss, ca, csum, sumdeg = _k3(s2, h2, dinv, b2r, Ws, bsr, degs)
    t = _seg_rows(s, ew, src, dst)
    out, oadj, spec, orth, clus = _k4(t, s, sth, ss, ca, csum, sumdeg)
    return (s, out, oadj, spec.reshape(()), orth.reshape(()),
            clus.reshape(()))


# speedup vs baseline: 6.3167x; 3.2682x over previous
import functools

import jax
import jax.numpy as jnp
from jax import lax
from jax.experimental import pallas as pl
from jax.experimental.pallas import tpu as pltpu

NN = 10000
EE = 320000
FF = 128
HH = 128
KK = 64
BLK = 2000
GRID = NN // BLK
HI = lax.Precision.DEFAULT
NP_ = 10240


def _selu(x):
    alpha = 1.6732632423543772848170429916717
    scale = 1.0507009873554804934193349852946
    return scale * jnp.where(x > 0, x, alpha * (jnp.exp(jnp.minimum(x, 0.0)) - 1.0))


def _k1_body(degp_ref, x_ref, w1_ref, h1_ref, u1_ref, dinv_ref):
    deg = degp_ref[0] + degp_ref[1] + 1.0
    dinv = lax.rsqrt(deg)
    h1 = jnp.dot(x_ref[...], w1_ref[...], precision=HI,
                 preferred_element_type=jnp.float32)
    h1_ref[...] = h1
    u1_ref[...] = h1 * dinv
    dinv_ref[...] = dinv


def _k1(degp3, x, W1):
    return pl.pallas_call(
        _k1_body,
        grid=(GRID,),
        in_specs=[
            pl.BlockSpec((2, BLK, 1), lambda i: (0, i, 0)),
            pl.BlockSpec((BLK, FF), lambda i: (i, 0)),
            pl.BlockSpec((FF, HH), lambda i: (0, 0)),
        ],
        out_specs=[
            pl.BlockSpec((BLK, HH), lambda i: (i, 0)),
            pl.BlockSpec((BLK, HH), lambda i: (i, 0)),
            pl.BlockSpec((BLK, 1), lambda i: (i, 0)),
        ],
        out_shape=[
            jax.ShapeDtypeStruct((NN, HH), jnp.float32),
            jax.ShapeDtypeStruct((NN, HH), jnp.float32),
            jax.ShapeDtypeStruct((NN, 1), jnp.float32),
        ],
    )(degp3, x, W1)


def _k2_body(sp_ref, h1_ref, dinv_ref, b1_ref, w2_ref, h2_ref, u2_ref):
    dinv = dinv_ref[...]
    agg = dinv * (sp_ref[0] + sp_ref[1]) + (dinv * dinv) * h1_ref[...]
    hsel = _selu(agg + b1_ref[...])
    h2 = jnp.dot(hsel, w2_ref[...], precision=HI,
                 preferred_element_type=jnp.float32)
    h2_ref[...] = h2
    u2_ref[...] = h2 * dinv


def _k2(sp3, h1, dinv, b1, W2):
    return pl.pallas_call(
        _k2_body,
        grid=(GRID,),
        in_specs=[
            pl.BlockSpec((2, BLK, HH), lambda i: (0, i, 0)),
            pl.BlockSpec((BLK, HH), lambda i: (i, 0)),
            pl.BlockSpec((BLK, 1), lambda i: (i, 0)),
            pl.BlockSpec((1, HH), lambda i: (0, 0)),
            pl.BlockSpec((HH, HH), lambda i: (0, 0)),
        ],
        out_specs=[
            pl.BlockSpec((BLK, HH), lambda i: (i, 0)),
            pl.BlockSpec((BLK, HH), lambda i: (i, 0)),
        ],
        out_shape=[
            jax.ShapeDtypeStruct((NN, HH), jnp.float32),
            jax.ShapeDtypeStruct((NN, HH), jnp.float32),
        ],
    )(sp3, h1, dinv, b1, W2)


def _k3_body(sp_ref, h2_ref, dinv_ref, b2_ref, ws_ref, bs_ref, degs_ref,
             s_ref, s128_ref, sth_ref, ss_ref, ca_ref, csum_ref, sumdeg_ref):
    i = pl.program_id(0)

    @pl.when(i == 0)
    def _():
        sth_ref[...] = jnp.zeros_like(sth_ref)
        ss_ref[...] = jnp.zeros_like(ss_ref)
        ca_ref[...] = jnp.zeros_like(ca_ref)
        csum_ref[...] = jnp.zeros_like(csum_ref)
        sumdeg_ref[...] = jnp.zeros_like(sumdeg_ref)

    dinv = dinv_ref[...]
    agg = dinv * (sp_ref[0] + sp_ref[1]) + (dinv * dinv) * h2_ref[...]
    hf = _selu(agg + b2_ref[...])
    logits = jnp.dot(hf, ws_ref[...], precision=HI,
                     preferred_element_type=jnp.float32) + bs_ref[...]
    z = logits - jnp.max(logits, axis=-1, keepdims=True)
    ez = jnp.exp(z)
    sblk = ez / jnp.sum(ez, axis=-1, keepdims=True)
    s_ref[...] = sblk
    s128_ref[...] = jnp.concatenate(
        [sblk, jnp.zeros((BLK, HH - KK), jnp.float32)], axis=1)
    dot_t = functools.partial(lax.dot_general,
                              dimension_numbers=(((0,), (0,)), ((), ())),
                              precision=HI,
                              preferred_element_type=jnp.float32)
    sth_ref[...] += dot_t(sblk, hf)
    ss_ref[...] += dot_t(sblk, sblk)
    degs = degs_ref[0] + degs_ref[1]
    ca_ref[...] += dot_t(degs, sblk)
    csum_ref[...] += jnp.sum(sblk, axis=0, keepdims=True)
    sumdeg_ref[...] += jnp.sum(degs, keepdims=True).reshape(1, 1)


def _k3(sp3, h2, dinv, b2, Ws, bs, degs):
    return pl.pallas_call(
        _k3_body,
        grid=(GRID,),
        in_specs=[
            pl.BlockSpec((2, BLK, HH), lambda i: (0, i, 0)),
            pl.BlockSpec((BLK, HH), lambda i: (i, 0)),
            pl.BlockSpec((BLK, 1), lambda i: (i, 0)),
            pl.BlockSpec((1, HH), lambda i: (0, 0)),
            pl.BlockSpec((HH, KK), lambda i: (0, 0)),
            pl.BlockSpec((1, KK), lambda i: (0, 0)),
            pl.BlockSpec((2, BLK, 1), lambda i: (0, i, 0)),
        ],
        out_specs=[
            pl.BlockSpec((BLK, KK), lambda i: (i, 0)),
            pl.BlockSpec((BLK, HH), lambda i: (i, 0)),
            pl.BlockSpec((KK, HH), lambda i: (0, 0)),
            pl.BlockSpec((KK, KK), lambda i: (0, 0)),
            pl.BlockSpec((1, KK), lambda i: (0, 0)),
            pl.BlockSpec((1, KK), lambda i: (0, 0)),
            pl.BlockSpec((1, 1), lambda i: (0, 0)),
        ],
        out_shape=[
            jax.ShapeDtypeStruct((NN, KK), jnp.float32),
            jax.ShapeDtypeStruct((NN, HH), jnp.float32),
            jax.ShapeDtypeStruct((KK, HH), jnp.float32),
            jax.ShapeDtypeStruct((KK, KK), jnp.float32),
            jax.ShapeDtypeStruct((1, KK), jnp.float32),
            jax.ShapeDtypeStruct((1, KK), jnp.float32),
            jax.ShapeDtypeStruct((1, 1), jnp.float32),
        ],
    )(sp3, h2, dinv, b2, Ws, bs, degs)


def _k4_body(tp_ref, s_ref, sth_ref, ss_ref, ca_ref, csum_ref, sumdeg_ref,
             out_ref, oadj_ref, spec_ref, orth_ref, clus_ref):
    i = pl.program_id(0)

    @pl.when(i == 0)
    def _():
        oadj_ref[...] = jnp.zeros_like(oadj_ref)

    t = tp_ref[0, :, :KK] + tp_ref[1, :, :KK]
    oadj_ref[...] += lax.dot_general(
        t, s_ref[...], dimension_numbers=(((0,), (0,)), ((), ())),
        precision=HI, preferred_element_type=jnp.float32)

    @pl.when(i == GRID - 1)
    def _():
        out_ref[...] = _selu(sth_ref[...])
        oadj = oadj_ref[...]
        m = sumdeg_ref[0, 0] / 2.0
        eye = (lax.broadcasted_iota(jnp.int32, (KK, KK), 0) ==
               lax.broadcasted_iota(jnp.int32, (KK, KK), 1)).astype(jnp.float32)
        ca = ca_ref[...]
        tr_oadj = jnp.sum(oadj * eye)
        tr_norm = jnp.sum(ca * ca) / (2.0 * m)
        spec_ref[...] = (-(tr_oadj - tr_norm) / (2.0 * m)).reshape(1, 1)
        ss = ss_ref[...]
        nss = jnp.sqrt(jnp.sum(ss * ss))
        d = ss / nss - eye / 8.0
        orth_ref[...] = jnp.sqrt(jnp.sum(d * d)).reshape(1, 1)
        csum = csum_ref[...]
        clus_ref[...] = (jnp.sqrt(jnp.sum(csum * csum)) / NN * 8.0
                         - 1.0).reshape(1, 1)


def _k4(tp3, s, sth, ss, ca, csum, sumdeg):
    return pl.pallas_call(
        _k4_body,
        grid=(GRID,),
        in_specs=[
            pl.BlockSpec((2, BLK, HH), lambda i: (0, i, 0)),
            pl.BlockSpec((BLK, KK), lambda i: (i, 0)),
            pl.BlockSpec((KK, HH), lambda i: (0, 0)),
            pl.BlockSpec((KK, KK), lambda i: (0, 0)),
            pl.BlockSpec((1, KK), lambda i: (0, 0)),
            pl.BlockSpec((1, KK), lambda i: (0, 0)),
            pl.BlockSpec((1, 1), lambda i: (0, 0)),
        ],
        out_specs=[
            pl.BlockSpec((KK, HH), lambda i: (0, 0)),
            pl.BlockSpec((KK, KK), lambda i: (0, 0)),
            pl.BlockSpec((1, 1), lambda i: (0, 0)),
            pl.BlockSpec((1, 1), lambda i: (0, 0)),
            pl.BlockSpec((1, 1), lambda i: (0, 0)),
        ],
        out_shape=[
            jax.ShapeDtypeStruct((KK, HH), jnp.float32),
            jax.ShapeDtypeStruct((KK, KK), jnp.float32),
            jax.ShapeDtypeStruct((1, 1), jnp.float32),
            jax.ShapeDtypeStruct((1, 1), jnp.float32),
            jax.ShapeDtypeStruct((1, 1), jnp.float32),
        ],
    )(tp3, s, sth, ss, ca, csum, sumdeg)


from jax.experimental.pallas import tpu_sc as plsc

CH = 80
EPT = EE // 32
NCH = EPT // CH
STRIPE = 640
_sc_mesh = plsc.VectorSubcoreMesh(core_axis_name="c", subcore_axis_name="s")


def _bcast16(v, i):
    idx = jnp.full((16, 1), i, jnp.int32)
    return lax.gather(
        v, idx,
        lax.GatherDimensionNumbers(offset_dims=(), collapsed_slice_dims=(0,),
                                   start_index_map=(0,)),
        (1,), mode=lax.GatherScatterMode.PROMISE_IN_BOUNDS)


@functools.partial(
    pl.kernel, mesh=_sc_mesh,
    out_type=[jax.ShapeDtypeStruct((2, NP_), jnp.float32),
              jax.ShapeDtypeStruct((2, NP_), jnp.float32)],
    scratch_types=[pltpu.VMEM((CH,), jnp.int32),
                   pltpu.VMEM((CH,), jnp.int32),
                   pltpu.VMEM((CH,), jnp.float32),
                   pltpu.VMEM((STRIPE,), jnp.float32),
                   pltpu.VMEM_SHARED((NP_,), jnp.float32),
                   pltpu.VMEM_SHARED((NP_,), jnp.float32)])
def _sc_deg(src_hbm, dst_hbm, ew_hbm, degd_hbm, degs_hbm,
            idxs_v, idxd_v, w_v, zb_v, degd_sh, degs_sh):
    cid = lax.axis_index("c")
    sid = lax.axis_index("s")

    @pl.loop(0, STRIPE // 16)
    def _(i):
        zb_v[pl.ds(i * 16, 16)] = jnp.zeros((16,), jnp.float32)

    pltpu.sync_copy(zb_v, degd_sh.at[pl.ds(sid * STRIPE, STRIPE)])
    pltpu.sync_copy(zb_v, degs_sh.at[pl.ds(sid * STRIPE, STRIPE)])
    plsc.subcore_barrier()
    base = (cid * 16 + sid) * EPT

    @pl.loop(0, NCH)
    def _(k):
        off = base + k * CH
        pltpu.sync_copy(src_hbm.at[pl.ds(off, CH)], idxs_v)
        pltpu.sync_copy(dst_hbm.at[pl.ds(off, CH)], idxd_v)
        pltpu.sync_copy(ew_hbm.at[pl.ds(off, CH)], w_v)
        pltpu.sync_copy(w_v, degd_sh.at[idxd_v], add=True)
        pltpu.sync_copy(w_v, degs_sh.at[idxs_v], add=True)

    plsc.subcore_barrier()
    pltpu.sync_copy(degd_sh.at[pl.ds(sid * STRIPE, STRIPE)],
                    degd_hbm.at[cid].at[pl.ds(sid * STRIPE, STRIPE)])
    pltpu.sync_copy(degs_sh.at[pl.ds(sid * STRIPE, STRIPE)],
                    degs_hbm.at[cid].at[pl.ds(sid * STRIPE, STRIPE)])


def _make_sc_seg(D):
    @functools.partial(
        pl.kernel, mesh=_sc_mesh,
        out_type=jax.ShapeDtypeStruct((2, NP_, D), jnp.float32),
        scratch_types=[pltpu.VMEM((CH,), jnp.int32),
                       pltpu.VMEM((CH,), jnp.int32),
                       pltpu.VMEM((CH,), jnp.float32),
                       pltpu.VMEM((CH, D), jnp.float32),
                       pltpu.VMEM_SHARED((NP_, D), jnp.float32)])
    def _sc_seg(u_hbm, src_hbm, dst_hbm, ew_hbm, out_hbm,
                idxs_v, idxd_v, w_v, rows_v, agg_sh):
        cid = lax.axis_index("c")
        sid = lax.axis_index("s")

        @pl.loop(0, CH)
        def _(r):
            for j in range(D // 16):
                rows_v[r, pl.ds(j * 16, 16)] = jnp.zeros((16,), jnp.float32)

        for t in range(STRIPE // CH):
            pltpu.sync_copy(rows_v,
                            agg_sh.at[pl.ds(sid * STRIPE + t * CH, CH)])
        plsc.subcore_barrier()
        base = (cid * 16 + sid) * EPT

        @pl.loop(0, NCH)
        def _(k):
            off = base + k * CH
            pltpu.sync_copy(src_hbm.at[pl.ds(off, CH)], idxs_v)
            pltpu.sync_copy(dst_hbm.at[pl.ds(off, CH)], idxd_v)
            pltpu.sync_copy(ew_hbm.at[pl.ds(off, CH)], w_v)
            pltpu.sync_copy(u_hbm.at[idxs_v], rows_v)

            @pl.loop(0, CH // 16)
            def _(g):
                w16 = w_v[pl.ds(g * 16, 16)]
                for i in range(16):
                    wb = _bcast16(w16, i)
                    r = g * 16 + i
                    for j in range(D // 16):
                        sl = pl.ds(j * 16, 16)
                        rows_v[r, sl] = rows_v[r, sl] * wb

            pltpu.sync_copy(rows_v, agg_sh.at[idxd_v], add=True)

        plsc.subcore_barrier()
        pltpu.sync_copy(agg_sh.at[pl.ds(sid * STRIPE, STRIPE)],
                        out_hbm.at[cid].at[pl.ds(sid * STRIPE, STRIPE)])

    return _sc_seg


_sc_seg128 = _make_sc_seg(HH)


def _seg_scalar(ew, src, dst):
    degd_p, degs_p = _sc_deg(src, dst, ew)
    return degd_p[:, :, None], degs_p[:, :, None]


def _seg_rows(u, ew, src, dst):
    return _sc_seg128(u, src, dst, ew)


def kernel(x, edge_index, edge_weight, W1, b1, W2, b2, Ws, bs):
    src = edge_index[0]
    dst = edge_index[1]
    ew = edge_weight
    b1r = b1.reshape(1, HH)
    b2r = b2.reshape(1, HH)
    bsr = bs.reshape(1, KK)

    degp3, degs = _seg_scalar(ew, src, dst)
    h1, u1, dinv = _k1(degp3, x, W1)
    s1 = _seg_rows(u1, ew, src, dst)
    h2, u2 = _k2(s1, h1, dinv, b1r, W2)
    s2 = _seg_rows(u2, ew, src, dst)
    s, s128, sth, ss, ca, csum, sumdeg = _k3(s2, h2, dinv, b2r, Ws, bsr, degs)
    t = _seg_rows(s128, ew, src, dst)
    out, oadj, spec, orth, clus = _k4(t, s, sth, ss, ca, csum, sumdeg)
    return (s, out, oadj, spec.reshape(()), orth.reshape(()),
            clus.reshape(()))


# speedup vs baseline: 18.1634x; 2.8754x over previous
import functools

import jax
import jax.numpy as jnp
from jax import lax
from jax.experimental import pallas as pl
from jax.experimental.pallas import tpu as pltpu

NN = 10000
EE = 320000
FF = 128
HH = 128
KK = 64
BLK = 2000
GRID = NN // BLK
HI = lax.Precision.DEFAULT
NP_ = 10240


def _selu(x):
    alpha = 1.6732632423543772848170429916717
    scale = 1.0507009873554804934193349852946
    return scale * jnp.where(x > 0, x, alpha * (jnp.exp(jnp.minimum(x, 0.0)) - 1.0))


def _k1_body(degp_ref, x_ref, w1_ref, h1_ref, u1_ref, dinv_ref):
    deg = degp_ref[0] + degp_ref[1] + 1.0
    dinv = lax.rsqrt(deg)
    h1 = jnp.dot(x_ref[...], w1_ref[...], precision=HI,
                 preferred_element_type=jnp.float32)
    h1_ref[...] = h1
    u1_ref[...] = h1 * dinv
    dinv_ref[...] = dinv


def _k1(degp3, x, W1):
    return pl.pallas_call(
        _k1_body,
        grid=(GRID,),
        in_specs=[
            pl.BlockSpec((2, BLK, 1), lambda i: (0, i, 0)),
            pl.BlockSpec((BLK, FF), lambda i: (i, 0)),
            pl.BlockSpec((FF, HH), lambda i: (0, 0)),
        ],
        out_specs=[
            pl.BlockSpec((BLK, HH), lambda i: (i, 0)),
            pl.BlockSpec((BLK, HH), lambda i: (i, 0)),
            pl.BlockSpec((BLK, 1), lambda i: (i, 0)),
        ],
        out_shape=[
            jax.ShapeDtypeStruct((NN, HH), jnp.float32),
            jax.ShapeDtypeStruct((NN, HH), jnp.float32),
            jax.ShapeDtypeStruct((NN, 1), jnp.float32),
        ],
    )(degp3, x, W1)


def _k2_body(sp_ref, h1_ref, dinv_ref, b1_ref, w2_ref, h2_ref, u2_ref):
    dinv = dinv_ref[...]
    agg = dinv * (sp_ref[0] + sp_ref[1]) + (dinv * dinv) * h1_ref[...]
    hsel = _selu(agg + b1_ref[...])
    h2 = jnp.dot(hsel, w2_ref[...], precision=HI,
                 preferred_element_type=jnp.float32)
    h2_ref[...] = h2
    u2_ref[...] = h2 * dinv


def _k2(sp3, h1, dinv, b1, W2):
    return pl.pallas_call(
        _k2_body,
        grid=(GRID,),
        in_specs=[
            pl.BlockSpec((2, BLK, HH), lambda i: (0, i, 0)),
            pl.BlockSpec((BLK, HH), lambda i: (i, 0)),
            pl.BlockSpec((BLK, 1), lambda i: (i, 0)),
            pl.BlockSpec((1, HH), lambda i: (0, 0)),
            pl.BlockSpec((HH, HH), lambda i: (0, 0)),
        ],
        out_specs=[
            pl.BlockSpec((BLK, HH), lambda i: (i, 0)),
            pl.BlockSpec((BLK, HH), lambda i: (i, 0)),
        ],
        out_shape=[
            jax.ShapeDtypeStruct((NN, HH), jnp.float32),
            jax.ShapeDtypeStruct((NN, HH), jnp.float32),
        ],
    )(sp3, h1, dinv, b1, W2)


def _k3_body(sp_ref, h2_ref, dinv_ref, b2_ref, ws_ref, bs_ref, degs_ref,
             s_ref, s128_ref, sth_ref, ss_ref, ca_ref, csum_ref, sumdeg_ref):
    i = pl.program_id(0)

    @pl.when(i == 0)
    def _():
        sth_ref[...] = jnp.zeros_like(sth_ref)
        ss_ref[...] = jnp.zeros_like(ss_ref)
        ca_ref[...] = jnp.zeros_like(ca_ref)
        csum_ref[...] = jnp.zeros_like(csum_ref)
        sumdeg_ref[...] = jnp.zeros_like(sumdeg_ref)

    dinv = dinv_ref[...]
    agg = dinv * (sp_ref[0] + sp_ref[1]) + (dinv * dinv) * h2_ref[...]
    hf = _selu(agg + b2_ref[...])
    logits = jnp.dot(hf, ws_ref[...], precision=HI,
                     preferred_element_type=jnp.float32) + bs_ref[...]
    z = logits - jnp.max(logits, axis=-1, keepdims=True)
    ez = jnp.exp(z)
    sblk = ez / jnp.sum(ez, axis=-1, keepdims=True)
    s_ref[...] = sblk
    s128_ref[...] = jnp.concatenate(
        [sblk, jnp.zeros((BLK, HH - KK), jnp.float32)], axis=1)
    dot_t = functools.partial(lax.dot_general,
                              dimension_numbers=(((0,), (0,)), ((), ())),
                              precision=HI,
                              preferred_element_type=jnp.float32)
    sth_ref[...] += dot_t(sblk, hf)
    ss_ref[...] += dot_t(sblk, sblk)
    degs = degs_ref[0] + degs_ref[1]
    ca_ref[...] += dot_t(degs, sblk)
    csum_ref[...] += jnp.sum(sblk, axis=0, keepdims=True)
    sumdeg_ref[...] += jnp.sum(degs, keepdims=True).reshape(1, 1)


def _k3(sp3, h2, dinv, b2, Ws, bs, degs):
    return pl.pallas_call(
        _k3_body,
        grid=(GRID,),
        in_specs=[
            pl.BlockSpec((2, BLK, HH), lambda i: (0, i, 0)),
            pl.BlockSpec((BLK, HH), lambda i: (i, 0)),
            pl.BlockSpec((BLK, 1), lambda i: (i, 0)),
            pl.BlockSpec((1, HH), lambda i: (0, 0)),
            pl.BlockSpec((HH, KK), lambda i: (0, 0)),
            pl.BlockSpec((1, KK), lambda i: (0, 0)),
            pl.BlockSpec((2, BLK, 1), lambda i: (0, i, 0)),
        ],
        out_specs=[
            pl.BlockSpec((BLK, KK), lambda i: (i, 0)),
            pl.BlockSpec((BLK, HH), lambda i: (i, 0)),
            pl.BlockSpec((KK, HH), lambda i: (0, 0)),
            pl.BlockSpec((KK, KK), lambda i: (0, 0)),
            pl.BlockSpec((1, KK), lambda i: (0, 0)),
            pl.BlockSpec((1, KK), lambda i: (0, 0)),
            pl.BlockSpec((1, 1), lambda i: (0, 0)),
        ],
        out_shape=[
            jax.ShapeDtypeStruct((NN, KK), jnp.float32),
            jax.ShapeDtypeStruct((NN, HH), jnp.float32),
            jax.ShapeDtypeStruct((KK, HH), jnp.float32),
            jax.ShapeDtypeStruct((KK, KK), jnp.float32),
            jax.ShapeDtypeStruct((1, KK), jnp.float32),
            jax.ShapeDtypeStruct((1, KK), jnp.float32),
            jax.ShapeDtypeStruct((1, 1), jnp.float32),
        ],
    )(sp3, h2, dinv, b2, Ws, bs, degs)


def _k4_body(tp_ref, s_ref, sth_ref, ss_ref, ca_ref, csum_ref, sumdeg_ref,
             out_ref, oadj_ref, spec_ref, orth_ref, clus_ref):
    i = pl.program_id(0)

    @pl.when(i == 0)
    def _():
        oadj_ref[...] = jnp.zeros_like(oadj_ref)

    t = tp_ref[0, :, :KK] + tp_ref[1, :, :KK]
    oadj_ref[...] += lax.dot_general(
        t, s_ref[...], dimension_numbers=(((0,), (0,)), ((), ())),
        precision=HI, preferred_element_type=jnp.float32)

    @pl.when(i == GRID - 1)
    def _():
        out_ref[...] = _selu(sth_ref[...])
        oadj = oadj_ref[...]
        m = sumdeg_ref[0, 0] / 2.0
        eye = (lax.broadcasted_iota(jnp.int32, (KK, KK), 0) ==
               lax.broadcasted_iota(jnp.int32, (KK, KK), 1)).astype(jnp.float32)
        ca = ca_ref[...]
        tr_oadj = jnp.sum(oadj * eye)
        tr_norm = jnp.sum(ca * ca) / (2.0 * m)
        spec_ref[...] = (-(tr_oadj - tr_norm) / (2.0 * m)).reshape(1, 1)
        ss = ss_ref[...]
        nss = jnp.sqrt(jnp.sum(ss * ss))
        d = ss / nss - eye / 8.0
        orth_ref[...] = jnp.sqrt(jnp.sum(d * d)).reshape(1, 1)
        csum = csum_ref[...]
        clus_ref[...] = (jnp.sqrt(jnp.sum(csum * csum)) / NN * 8.0
                         - 1.0).reshape(1, 1)


def _k4(tp3, s, sth, ss, ca, csum, sumdeg):
    return pl.pallas_call(
        _k4_body,
        grid=(GRID,),
        in_specs=[
            pl.BlockSpec((2, BLK, HH), lambda i: (0, i, 0)),
            pl.BlockSpec((BLK, KK), lambda i: (i, 0)),
            pl.BlockSpec((KK, HH), lambda i: (0, 0)),
            pl.BlockSpec((KK, KK), lambda i: (0, 0)),
            pl.BlockSpec((1, KK), lambda i: (0, 0)),
            pl.BlockSpec((1, KK), lambda i: (0, 0)),
            pl.BlockSpec((1, 1), lambda i: (0, 0)),
        ],
        out_specs=[
            pl.BlockSpec((KK, HH), lambda i: (0, 0)),
            pl.BlockSpec((KK, KK), lambda i: (0, 0)),
            pl.BlockSpec((1, 1), lambda i: (0, 0)),
            pl.BlockSpec((1, 1), lambda i: (0, 0)),
            pl.BlockSpec((1, 1), lambda i: (0, 0)),
        ],
        out_shape=[
            jax.ShapeDtypeStruct((KK, HH), jnp.float32),
            jax.ShapeDtypeStruct((KK, KK), jnp.float32),
            jax.ShapeDtypeStruct((1, 1), jnp.float32),
            jax.ShapeDtypeStruct((1, 1), jnp.float32),
            jax.ShapeDtypeStruct((1, 1), jnp.float32),
        ],
    )(tp3, s, sth, ss, ca, csum, sumdeg)


from jax.experimental.pallas import tpu_sc as plsc

CH = 80
EPT = EE // 32
NCH = EPT // CH
STRIPE = 640
_sc_mesh = plsc.VectorSubcoreMesh(core_axis_name="c", subcore_axis_name="s")


def _bcast16(v, i):
    idx = jnp.full((16, 1), i, jnp.int32)
    return lax.gather(
        v, idx,
        lax.GatherDimensionNumbers(offset_dims=(), collapsed_slice_dims=(0,),
                                   start_index_map=(0,)),
        (1,), mode=lax.GatherScatterMode.PROMISE_IN_BOUNDS)


NBUF = 4
LAG = 3
NMAIN = (NCH // NBUF) * NBUF


@functools.partial(
    pl.kernel, mesh=_sc_mesh,
    out_type=[jax.ShapeDtypeStruct((2, NP_), jnp.float32),
              jax.ShapeDtypeStruct((2, NP_), jnp.float32)],
    scratch_types=[pltpu.VMEM((NBUF, CH), jnp.int32),
                   pltpu.VMEM((NBUF, CH), jnp.int32),
                   pltpu.VMEM((NBUF, CH), jnp.float32),
                   pltpu.VMEM((STRIPE,), jnp.float32),
                   pltpu.VMEM_SHARED((NP_,), jnp.float32),
                   pltpu.VMEM_SHARED((NP_,), jnp.float32),
                   pltpu.SemaphoreType.DMA((NBUF,)),
                   pltpu.SemaphoreType.DMA((NBUF,)),
                   pltpu.SemaphoreType.DMA((NBUF,))])
def _sc_deg(src_hbm, dst_hbm, ew_hbm, degd_hbm, degs_hbm,
            idxs_v, idxd_v, w_v, zb_v, degd_sh, degs_sh,
            semL, semSd, semSs):
    cid = lax.axis_index("c")
    sid = lax.axis_index("s")

    @pl.loop(0, STRIPE // 16)
    def _(i):
        zb_v[pl.ds(i * 16, 16)] = jnp.zeros((16,), jnp.float32)

    pltpu.sync_copy(zb_v, degd_sh.at[pl.ds(sid * STRIPE, STRIPE)])
    pltpu.sync_copy(zb_v, degs_sh.at[pl.ds(sid * STRIPE, STRIPE)])
    plsc.subcore_barrier()
    base = (cid * 16 + sid) * EPT

    def issue_loads(j, c):
        off = base + c * CH
        pltpu.async_copy(src_hbm.at[pl.ds(off, CH)], idxs_v.at[j], semL.at[j])
        pltpu.async_copy(dst_hbm.at[pl.ds(off, CH)], idxd_v.at[j], semL.at[j])
        pltpu.async_copy(ew_hbm.at[pl.ds(off, CH)], w_v.at[j], semL.at[j])

    def wait_loads(j):
        pltpu.make_async_copy(src_hbm.at[pl.ds(0, CH)], idxs_v.at[j],
                              semL.at[j]).wait()
        pltpu.make_async_copy(dst_hbm.at[pl.ds(0, CH)], idxd_v.at[j],
                              semL.at[j]).wait()
        pltpu.make_async_copy(ew_hbm.at[pl.ds(0, CH)], w_v.at[j],
                              semL.at[j]).wait()

    def wait_scatters(j):
        pltpu.make_async_copy(w_v.at[j], degd_sh.at[idxd_v.at[j]],
                              semSd.at[j]).wait()
        pltpu.make_async_copy(w_v.at[j], degs_sh.at[idxs_v.at[j]],
                              semSs.at[j]).wait()

    def item(j, c):
        wait_loads(j)
        pltpu.async_copy(w_v.at[j], degd_sh.at[idxd_v.at[j]],
                         semSd.at[j], add=True)
        pltpu.async_copy(w_v.at[j], degs_sh.at[idxs_v.at[j]],
                         semSs.at[j], add=True)
        jp = (j + LAG) % NBUF
        cp = c + LAG

        @pl.when(cp < NCH)
        def _():
            @pl.when(cp >= NBUF)
            def _():
                wait_scatters(jp)

            issue_loads(jp, cp)

    for j in range(LAG):
        issue_loads(j, j)

    @pl.loop(0, NMAIN // NBUF)
    def _(t):
        for j in range(NBUF):
            item(j, t * NBUF + j)

    for c in range(NMAIN, NCH):
        item(c % NBUF, c)

    for j in range(NBUF):
        wait_scatters(j)
    plsc.subcore_barrier()
    pltpu.sync_copy(degd_sh.at[pl.ds(sid * STRIPE, STRIPE)],
                    degd_hbm.at[cid].at[pl.ds(sid * STRIPE, STRIPE)])
    pltpu.sync_copy(degs_sh.at[pl.ds(sid * STRIPE, STRIPE)],
                    degs_hbm.at[cid].at[pl.ds(sid * STRIPE, STRIPE)])


def _make_sc_seg(D):
    @functools.partial(
        pl.kernel, mesh=_sc_mesh,
        out_type=jax.ShapeDtypeStruct((2, NP_, D), jnp.float32),
        scratch_types=[pltpu.VMEM((NBUF, CH), jnp.int32),
                       pltpu.VMEM((NBUF, CH), jnp.int32),
                       pltpu.VMEM((NBUF, CH), jnp.float32),
                       pltpu.VMEM((NBUF, CH, D), jnp.float32),
                       pltpu.VMEM_SHARED((NP_, D), jnp.float32),
                       pltpu.SemaphoreType.DMA((NBUF,)),
                       pltpu.SemaphoreType.DMA((NBUF,)),
                       pltpu.SemaphoreType.DMA((NBUF,))])
    def _sc_seg(u_hbm, src_hbm, dst_hbm, ew_hbm, out_hbm,
                idxs_v, idxd_v, w_v, rows_v, agg_sh, semL, semG, semS):
        cid = lax.axis_index("c")
        sid = lax.axis_index("s")

        @pl.loop(0, CH)
        def _(r):
            for j in range(D // 16):
                rows_v[0, r, pl.ds(j * 16, 16)] = jnp.zeros((16,), jnp.float32)

        for t in range(STRIPE // CH):
            pltpu.sync_copy(rows_v.at[0],
                            agg_sh.at[pl.ds(sid * STRIPE + t * CH, CH)])
        plsc.subcore_barrier()
        base = (cid * 16 + sid) * EPT

        def issue_loads(j, c):
            off = base + c * CH
            pltpu.async_copy(src_hbm.at[pl.ds(off, CH)], idxs_v.at[j],
                             semL.at[j])
            pltpu.async_copy(dst_hbm.at[pl.ds(off, CH)], idxd_v.at[j],
                             semL.at[j])
            pltpu.async_copy(ew_hbm.at[pl.ds(off, CH)], w_v.at[j], semL.at[j])

        def wait_loads(j):
            pltpu.make_async_copy(src_hbm.at[pl.ds(0, CH)], idxs_v.at[j],
                                  semL.at[j]).wait()
            pltpu.make_async_copy(dst_hbm.at[pl.ds(0, CH)], idxd_v.at[j],
                                  semL.at[j]).wait()
            pltpu.make_async_copy(ew_hbm.at[pl.ds(0, CH)], w_v.at[j],
                                  semL.at[j]).wait()

        def issue_gather(j):
            pltpu.async_copy(u_hbm.at[idxs_v.at[j]], rows_v.at[j], semG.at[j])

        def wait_gather(j):
            pltpu.make_async_copy(u_hbm.at[idxs_v.at[j]], rows_v.at[j],
                                  semG.at[j]).wait()

        def wait_scatter(j):
            pltpu.make_async_copy(rows_v.at[j], agg_sh.at[idxd_v.at[j]],
                                  semS.at[j]).wait()

        def item(j, c):
            wait_gather(j)

            @pl.loop(0, CH // 16)
            def _(g):
                w16 = w_v[j, pl.ds(g * 16, 16)]
                for i in range(16):
                    wb = _bcast16(w16, i)
                    r = g * 16 + i
                    for q in range(D // 16):
                        sl = pl.ds(q * 16, 16)
                        rows_v[j, r, sl] = rows_v[j, r, sl] * wb

            pltpu.async_copy(rows_v.at[j], agg_sh.at[idxd_v.at[j]],
                             semS.at[j], add=True)
            jp = (j + LAG) % NBUF
            cp = c + LAG

            @pl.when(cp < NCH)
            def _():
                @pl.when(cp >= NBUF)
                def _():
                    wait_scatter(jp)

                issue_loads(jp, cp)
                wait_loads(jp)
                issue_gather(jp)

        for j in range(LAG):
            issue_loads(j, j)
            wait_loads(j)
            issue_gather(j)

        @pl.loop(0, NMAIN // NBUF)
        def _(t):
            for j in range(NBUF):
                item(j, t * NBUF + j)

        for c in range(NMAIN, NCH):
            item(c % NBUF, c)

        for j in range(NBUF):
            wait_scatter(j)
        plsc.subcore_barrier()
        pltpu.sync_copy(agg_sh.at[pl.ds(sid * STRIPE, STRIPE)],
                        out_hbm.at[cid].at[pl.ds(sid * STRIPE, STRIPE)])

    return _sc_seg


_sc_seg128 = _make_sc_seg(HH)


def _seg_scalar(ew, src, dst):
    degd_p, degs_p = _sc_deg(src, dst, ew)
    return degd_p[:, :, None], degs_p[:, :, None]


def _seg_rows(u, ew, src, dst):
    return _sc_seg128(u, src, dst, ew)


def kernel(x, edge_index, edge_weight, W1, b1, W2, b2, Ws, bs):
    src = edge_index[0]
    dst = edge_index[1]
    ew = edge_weight
    b1r = b1.reshape(1, HH)
    b2r = b2.reshape(1, HH)
    bsr = bs.reshape(1, KK)

    degp3, degs = _seg_scalar(ew, src, dst)
    h1, u1, dinv = _k1(degp3, x, W1)
    s1 = _seg_rows(u1, ew, src, dst)
    h2, u2 = _k2(s1, h1, dinv, b1r, W2)
    s2 = _seg_rows(u2, ew, src, dst)
    s, s128, sth, ss, ca, csum, sumdeg = _k3(s2, h2, dinv, b2r, Ws, bsr, degs)
    t = _seg_rows(s128, ew, src, dst)
    out, oadj, spec, orth, clus = _k4(t, s, sth, ss, ca, csum, sumdeg)
    return (s, out, oadj, spec.reshape(()), orth.reshape(()),
            clus.reshape(()))


# speedup vs baseline: 18.7823x; 1.0341x over previous
import functools

import jax
import jax.numpy as jnp
from jax import lax
from jax.experimental import pallas as pl
from jax.experimental.pallas import tpu as pltpu

NN = 10000
EE = 320000
FF = 128
HH = 128
KK = 64
BLK = 2000
GRID = NN // BLK
HI = lax.Precision.DEFAULT
NP_ = 10240


def _selu(x):
    alpha = 1.6732632423543772848170429916717
    scale = 1.0507009873554804934193349852946
    return scale * jnp.where(x > 0, x, alpha * (jnp.exp(jnp.minimum(x, 0.0)) - 1.0))


def _k1a_body(x_ref, w1_ref, h1_ref):
    h1_ref[...] = jnp.dot(x_ref[...], w1_ref[...], precision=HI,
                          preferred_element_type=jnp.float32)


def _k1a(x, W1):
    return pl.pallas_call(
        _k1a_body,
        grid=(GRID,),
        in_specs=[
            pl.BlockSpec((BLK, FF), lambda i: (i, 0)),
            pl.BlockSpec((FF, HH), lambda i: (0, 0)),
        ],
        out_specs=pl.BlockSpec((BLK, HH), lambda i: (i, 0)),
        out_shape=jax.ShapeDtypeStruct((NN, HH), jnp.float32),
    )(x, W1)


def _k1b_body(degp_ref, h1_ref, u1_ref, dinv_ref):
    deg = degp_ref[0] + degp_ref[1] + 1.0
    dinv = lax.rsqrt(deg)
    u1_ref[...] = h1_ref[...] * dinv
    dinv_ref[...] = dinv


def _k1b(degp3, h1):
    return pl.pallas_call(
        _k1b_body,
        grid=(GRID,),
        in_specs=[
            pl.BlockSpec((2, BLK, 1), lambda i: (0, i, 0)),
            pl.BlockSpec((BLK, HH), lambda i: (i, 0)),
        ],
        out_specs=[
            pl.BlockSpec((BLK, HH), lambda i: (i, 0)),
            pl.BlockSpec((BLK, 1), lambda i: (i, 0)),
        ],
        out_shape=[
            jax.ShapeDtypeStruct((NN, HH), jnp.float32),
            jax.ShapeDtypeStruct((NN, 1), jnp.float32),
        ],
    )(degp3, h1)


def _k2_body(sp_ref, h1_ref, dinv_ref, b1_ref, w2_ref, h2_ref, u2_ref):
    dinv = dinv_ref[...]
    agg = dinv * (sp_ref[0] + sp_ref[1]) + (dinv * dinv) * h1_ref[...]
    hsel = _selu(agg + b1_ref[...])
    h2 = jnp.dot(hsel, w2_ref[...], precision=HI,
                 preferred_element_type=jnp.float32)
    h2_ref[...] = h2
    u2_ref[...] = h2 * dinv


def _k2(sp3, h1, dinv, b1, W2):
    return pl.pallas_call(
        _k2_body,
        grid=(GRID,),
        in_specs=[
            pl.BlockSpec((2, BLK, HH), lambda i: (0, i, 0)),
            pl.BlockSpec((BLK, HH), lambda i: (i, 0)),
            pl.BlockSpec((BLK, 1), lambda i: (i, 0)),
            pl.BlockSpec((1, HH), lambda i: (0, 0)),
            pl.BlockSpec((HH, HH), lambda i: (0, 0)),
        ],
        out_specs=[
            pl.BlockSpec((BLK, HH), lambda i: (i, 0)),
            pl.BlockSpec((BLK, HH), lambda i: (i, 0)),
        ],
        out_shape=[
            jax.ShapeDtypeStruct((NN, HH), jnp.float32),
            jax.ShapeDtypeStruct((NN, HH), jnp.float32),
        ],
    )(sp3, h1, dinv, b1, W2)


def _k3_body(sp_ref, h2_ref, dinv_ref, b2_ref, ws_ref, bs_ref, degs_ref,
             s_ref, s128_ref, sth_ref, ss_ref, ca_ref, csum_ref, sumdeg_ref):
    i = pl.program_id(0)

    @pl.when(i == 0)
    def _():
        sth_ref[...] = jnp.zeros_like(sth_ref)
        ss_ref[...] = jnp.zeros_like(ss_ref)
        ca_ref[...] = jnp.zeros_like(ca_ref)
        csum_ref[...] = jnp.zeros_like(csum_ref)
        sumdeg_ref[...] = jnp.zeros_like(sumdeg_ref)

    dinv = dinv_ref[...]
    agg = dinv * (sp_ref[0] + sp_ref[1]) + (dinv * dinv) * h2_ref[...]
    hf = _selu(agg + b2_ref[...])
    logits = jnp.dot(hf, ws_ref[...], precision=HI,
                     preferred_element_type=jnp.float32) + bs_ref[...]
    z = logits - jnp.max(logits, axis=-1, keepdims=True)
    ez = jnp.exp(z)
    sblk = ez / jnp.sum(ez, axis=-1, keepdims=True)
    s_ref[...] = sblk
    s128_ref[...] = jnp.concatenate(
        [sblk, jnp.zeros((BLK, HH - KK), jnp.float32)], axis=1)
    dot_t = functools.partial(lax.dot_general,
                              dimension_numbers=(((0,), (0,)), ((), ())),
                              precision=HI,
                              preferred_element_type=jnp.float32)
    sth_ref[...] += dot_t(sblk, hf)
    ss_ref[...] += dot_t(sblk, sblk)
    degs = degs_ref[0] + degs_ref[1]
    ca_ref[...] += dot_t(degs, sblk)
    csum_ref[...] += jnp.sum(sblk, axis=0, keepdims=True)
    sumdeg_ref[...] += jnp.sum(degs, keepdims=True).reshape(1, 1)


def _k3(sp3, h2, dinv, b2, Ws, bs, degs):
    return pl.pallas_call(
        _k3_body,
        grid=(GRID,),
        in_specs=[
            pl.BlockSpec((2, BLK, HH), lambda i: (0, i, 0)),
            pl.BlockSpec((BLK, HH), lambda i: (i, 0)),
            pl.BlockSpec((BLK, 1), lambda i: (i, 0)),
            pl.BlockSpec((1, HH), lambda i: (0, 0)),
            pl.BlockSpec((HH, KK), lambda i: (0, 0)),
            pl.BlockSpec((1, KK), lambda i: (0, 0)),
            pl.BlockSpec((2, BLK, 1), lambda i: (0, i, 0)),
        ],
        out_specs=[
            pl.BlockSpec((BLK, KK), lambda i: (i, 0)),
            pl.BlockSpec((BLK, HH), lambda i: (i, 0)),
            pl.BlockSpec((KK, HH), lambda i: (0, 0)),
            pl.BlockSpec((KK, KK), lambda i: (0, 0)),
            pl.BlockSpec((1, KK), lambda i: (0, 0)),
            pl.BlockSpec((1, KK), lambda i: (0, 0)),
            pl.BlockSpec((1, 1), lambda i: (0, 0)),
        ],
        out_shape=[
            jax.ShapeDtypeStruct((NN, KK), jnp.float32),
            jax.ShapeDtypeStruct((NN, HH), jnp.float32),
            jax.ShapeDtypeStruct((KK, HH), jnp.float32),
            jax.ShapeDtypeStruct((KK, KK), jnp.float32),
            jax.ShapeDtypeStruct((1, KK), jnp.float32),
            jax.ShapeDtypeStruct((1, KK), jnp.float32),
            jax.ShapeDtypeStruct((1, 1), jnp.float32),
        ],
    )(sp3, h2, dinv, b2, Ws, bs, degs)


def _k4_body(tp_ref, s_ref, sth_ref, ss_ref, ca_ref, csum_ref, sumdeg_ref,
             out_ref, oadj_ref, spec_ref, orth_ref, clus_ref):
    i = pl.program_id(0)

    @pl.when(i == 0)
    def _():
        oadj_ref[...] = jnp.zeros_like(oadj_ref)

    t = tp_ref[0, :, :KK] + tp_ref[1, :, :KK]
    oadj_ref[...] += lax.dot_general(
        t, s_ref[...], dimension_numbers=(((0,), (0,)), ((), ())),
        precision=HI, preferred_element_type=jnp.float32)

    @pl.when(i == GRID - 1)
    def _():
        out_ref[...] = _selu(sth_ref[...])
        oadj = oadj_ref[...]
        m = sumdeg_ref[0, 0] / 2.0
        eye = (lax.broadcasted_iota(jnp.int32, (KK, KK), 0) ==
               lax.broadcasted_iota(jnp.int32, (KK, KK), 1)).astype(jnp.float32)
        ca = ca_ref[...]
        tr_oadj = jnp.sum(oadj * eye)
        tr_norm = jnp.sum(ca * ca) / (2.0 * m)
        spec_ref[...] = (-(tr_oadj - tr_norm) / (2.0 * m)).reshape(1, 1)
        ss = ss_ref[...]
        nss = jnp.sqrt(jnp.sum(ss * ss))
        d = ss / nss - eye / 8.0
        orth_ref[...] = jnp.sqrt(jnp.sum(d * d)).reshape(1, 1)
        csum = csum_ref[...]
        clus_ref[...] = (jnp.sqrt(jnp.sum(csum * csum)) / NN * 8.0
                         - 1.0).reshape(1, 1)


def _k4(tp3, s, sth, ss, ca, csum, sumdeg):
    return pl.pallas_call(
        _k4_body,
        grid=(GRID,),
        in_specs=[
            pl.BlockSpec((2, BLK, HH), lambda i: (0, i, 0)),
            pl.BlockSpec((BLK, KK), lambda i: (i, 0)),
            pl.BlockSpec((KK, HH), lambda i: (0, 0)),
            pl.BlockSpec((KK, KK), lambda i: (0, 0)),
            pl.BlockSpec((1, KK), lambda i: (0, 0)),
            pl.BlockSpec((1, KK), lambda i: (0, 0)),
            pl.BlockSpec((1, 1), lambda i: (0, 0)),
        ],
        out_specs=[
            pl.BlockSpec((KK, HH), lambda i: (0, 0)),
            pl.BlockSpec((KK, KK), lambda i: (0, 0)),
            pl.BlockSpec((1, 1), lambda i: (0, 0)),
            pl.BlockSpec((1, 1), lambda i: (0, 0)),
            pl.BlockSpec((1, 1), lambda i: (0, 0)),
        ],
        out_shape=[
            jax.ShapeDtypeStruct((KK, HH), jnp.float32),
            jax.ShapeDtypeStruct((KK, KK), jnp.float32),
            jax.ShapeDtypeStruct((1, 1), jnp.float32),
            jax.ShapeDtypeStruct((1, 1), jnp.float32),
            jax.ShapeDtypeStruct((1, 1), jnp.float32),
        ],
    )(tp3, s, sth, ss, ca, csum, sumdeg)


from jax.experimental.pallas import tpu_sc as plsc

CH = 80
EPT = EE // 32
NCH = EPT // CH
STRIPE = 640
_sc_mesh = plsc.VectorSubcoreMesh(core_axis_name="c", subcore_axis_name="s")


def _bcast16(v, i):
    idx = jnp.full((16, 1), i, jnp.int32)
    return lax.gather(
        v, idx,
        lax.GatherDimensionNumbers(offset_dims=(), collapsed_slice_dims=(0,),
                                   start_index_map=(0,)),
        (1,), mode=lax.GatherScatterMode.PROMISE_IN_BOUNDS)


NBUF = 4
LAG = 3
NMAIN = (NCH // NBUF) * NBUF


@functools.partial(
    pl.kernel, mesh=_sc_mesh,
    out_type=[jax.ShapeDtypeStruct((2, NP_), jnp.float32),
              jax.ShapeDtypeStruct((2, NP_), jnp.float32)],
    scratch_types=[pltpu.VMEM((NBUF, CH), jnp.int32),
                   pltpu.VMEM((NBUF, CH), jnp.int32),
                   pltpu.VMEM((NBUF, CH), jnp.float32),
                   pltpu.VMEM((STRIPE,), jnp.float32),
                   pltpu.VMEM_SHARED((NP_,), jnp.float32),
                   pltpu.VMEM_SHARED((NP_,), jnp.float32),
                   pltpu.SemaphoreType.DMA((NBUF,)),
                   pltpu.SemaphoreType.DMA((NBUF,)),
                   pltpu.SemaphoreType.DMA((NBUF,))])
def _sc_deg(src_hbm, dst_hbm, ew_hbm, degd_hbm, degs_hbm,
            idxs_v, idxd_v, w_v, zb_v, degd_sh, degs_sh,
            semL, semSd, semSs):
    cid = lax.axis_index("c")
    sid = lax.axis_index("s")

    @pl.loop(0, STRIPE // 16)
    def _(i):
        zb_v[pl.ds(i * 16, 16)] = jnp.zeros((16,), jnp.float32)

    pltpu.sync_copy(zb_v, degd_sh.at[pl.ds(sid * STRIPE, STRIPE)])
    pltpu.sync_copy(zb_v, degs_sh.at[pl.ds(sid * STRIPE, STRIPE)])
    plsc.subcore_barrier()
    base = (cid * 16 + sid) * EPT

    def issue_loads(j, c):
        off = base + c * CH
        pltpu.async_copy(src_hbm.at[pl.ds(off, CH)], idxs_v.at[j], semL.at[j])
        pltpu.async_copy(dst_hbm.at[pl.ds(off, CH)], idxd_v.at[j], semL.at[j])
        pltpu.async_copy(ew_hbm.at[pl.ds(off, CH)], w_v.at[j], semL.at[j])

    def wait_loads(j):
        pltpu.make_async_copy(src_hbm.at[pl.ds(0, CH)], idxs_v.at[j],
                              semL.at[j]).wait()
        pltpu.make_async_copy(dst_hbm.at[pl.ds(0, CH)], idxd_v.at[j],
                              semL.at[j]).wait()
        pltpu.make_async_copy(ew_hbm.at[pl.ds(0, CH)], w_v.at[j],
                              semL.at[j]).wait()

    def wait_scatters(j):
        pltpu.make_async_copy(w_v.at[j], degd_sh.at[idxd_v.at[j]],
                              semSd.at[j]).wait()
        pltpu.make_async_copy(w_v.at[j], degs_sh.at[idxs_v.at[j]],
                              semSs.at[j]).wait()

    def item(j, c):
        wait_loads(j)
        pltpu.async_copy(w_v.at[j], degd_sh.at[idxd_v.at[j]],
                         semSd.at[j], add=True)
        pltpu.async_copy(w_v.at[j], degs_sh.at[idxs_v.at[j]],
                         semSs.at[j], add=True)
        jp = (j + LAG) % NBUF
        cp = c + LAG

        @pl.when(cp < NCH)
        def _():
            @pl.when(cp >= NBUF)
            def _():
                wait_scatters(jp)

            issue_loads(jp, cp)

    for j in range(LAG):
        issue_loads(j, j)

    @pl.loop(0, NMAIN // NBUF)
    def _(t):
        for j in range(NBUF):
            item(j, t * NBUF + j)

    for c in range(NMAIN, NCH):
        item(c % NBUF, c)

    for j in range(NBUF):
        wait_scatters(j)
    plsc.subcore_barrier()
    pltpu.sync_copy(degd_sh.at[pl.ds(sid * STRIPE, STRIPE)],
                    degd_hbm.at[cid].at[pl.ds(sid * STRIPE, STRIPE)])
    pltpu.sync_copy(degs_sh.at[pl.ds(sid * STRIPE, STRIPE)],
                    degs_hbm.at[cid].at[pl.ds(sid * STRIPE, STRIPE)])


def _make_sc_seg(D, DA):
    @functools.partial(
        pl.kernel, mesh=_sc_mesh,
        out_type=jax.ShapeDtypeStruct((2, NP_, D), jnp.float32),
        scratch_types=[pltpu.VMEM((NBUF, CH), jnp.int32),
                       pltpu.VMEM((NBUF, CH), jnp.int32),
                       pltpu.VMEM((NBUF, CH), jnp.float32),
                       pltpu.VMEM((NBUF, CH, D), jnp.float32),
                       pltpu.VMEM_SHARED((NP_, D), jnp.float32),
                       pltpu.SemaphoreType.DMA((NBUF,)),
                       pltpu.SemaphoreType.DMA((NBUF,)),
                       pltpu.SemaphoreType.DMA((NBUF,))])
    def _sc_seg(u_hbm, src_hbm, dst_hbm, ew_hbm, out_hbm,
                idxs_v, idxd_v, w_v, rows_v, agg_sh, semL, semG, semS):
        cid = lax.axis_index("c")
        sid = lax.axis_index("s")

        @pl.loop(0, CH)
        def _(r):
            for j in range(D // 16):
                rows_v[0, r, pl.ds(j * 16, 16)] = jnp.zeros((16,), jnp.float32)

        for t in range(STRIPE // CH):
            pltpu.sync_copy(rows_v.at[0],
                            agg_sh.at[pl.ds(sid * STRIPE + t * CH, CH)])
        plsc.subcore_barrier()
        base = (cid * 16 + sid) * EPT

        def issue_loads(j, c):
            off = base + c * CH
            pltpu.async_copy(src_hbm.at[pl.ds(off, CH)], idxs_v.at[j],
                             semL.at[j])
            pltpu.async_copy(dst_hbm.at[pl.ds(off, CH)], idxd_v.at[j],
                             semL.at[j])
            pltpu.async_copy(ew_hbm.at[pl.ds(off, CH)], w_v.at[j], semL.at[j])

        def wait_loads(j):
            pltpu.make_async_copy(src_hbm.at[pl.ds(0, CH)], idxs_v.at[j],
                                  semL.at[j]).wait()
            pltpu.make_async_copy(dst_hbm.at[pl.ds(0, CH)], idxd_v.at[j],
                                  semL.at[j]).wait()
            pltpu.make_async_copy(ew_hbm.at[pl.ds(0, CH)], w_v.at[j],
                                  semL.at[j]).wait()

        def issue_gather(j):
            pltpu.async_copy(u_hbm.at[idxs_v.at[j]], rows_v.at[j], semG.at[j])

        def wait_gather(j):
            pltpu.make_async_copy(u_hbm.at[idxs_v.at[j]], rows_v.at[j],
                                  semG.at[j]).wait()

        def wait_scatter(j):
            pltpu.make_async_copy(rows_v.at[j], agg_sh.at[idxd_v.at[j]],
                                  semS.at[j]).wait()

        def item(j, c):
            wait_gather(j)

            @pl.loop(0, CH // 16)
            def _(g):
                w16 = w_v[j, pl.ds(g * 16, 16)]
                for i in range(16):
                    wb = _bcast16(w16, i)
                    r = g * 16 + i
                    for q in range(DA // 16):
                        sl = pl.ds(q * 16, 16)
                        rows_v[j, r, sl] = rows_v[j, r, sl] * wb

            pltpu.async_copy(rows_v.at[j], agg_sh.at[idxd_v.at[j]],
                             semS.at[j], add=True)
            jp = (j + LAG) % NBUF
            cp = c + LAG

            @pl.when(cp < NCH)
            def _():
                @pl.when(cp >= NBUF)
                def _():
                    wait_scatter(jp)

                issue_loads(jp, cp)
                wait_loads(jp)
                issue_gather(jp)

        for j in range(LAG):
            issue_loads(j, j)
            wait_loads(j)
            issue_gather(j)

        @pl.loop(0, NMAIN // NBUF)
        def _(t):
            for j in range(NBUF):
                item(j, t * NBUF + j)

        for c in range(NMAIN, NCH):
            item(c % NBUF, c)

        for j in range(NBUF):
            wait_scatter(j)
        plsc.subcore_barrier()
        pltpu.sync_copy(agg_sh.at[pl.ds(sid * STRIPE, STRIPE)],
                        out_hbm.at[cid].at[pl.ds(sid * STRIPE, STRIPE)])

    return _sc_seg


_sc_seg128 = _make_sc_seg(HH, HH)
_sc_seg_s = _make_sc_seg(HH, KK)


def _seg_scalar(ew, src, dst):
    degd_p, degs_p = _sc_deg(src, dst, ew)
    return degd_p[:, :, None], degs_p[:, :, None]


def _seg_rows(u, ew, src, dst):
    return _sc_seg128(u, src, dst, ew)


def kernel(x, edge_index, edge_weight, W1, b1, W2, b2, Ws, bs):
    src = edge_index[0]
    dst = edge_index[1]
    ew = edge_weight
    b1r = b1.reshape(1, HH)
    b2r = b2.reshape(1, HH)
    bsr = bs.reshape(1, KK)

    degp3, degs = _seg_scalar(ew, src, dst)
    h1 = _k1a(x, W1)
    u1, dinv = _k1b(degp3, h1)
    s1 = _seg_rows(u1, ew, src, dst)
    h2, u2 = _k2(s1, h1, dinv, b1r, W2)
    s2 = _seg_rows(u2, ew, src, dst)
    s, s128, sth, ss, ca, csum, sumdeg = _k3(s2, h2, dinv, b2r, Ws, bsr, degs)
    t = _sc_seg_s(s128, src, dst, ew)
    out, oadj, spec, orth, clus = _k4(t, s, sth, ss, ca, csum, sumdeg)
    return (s, out, oadj, spec.reshape(()), orth.reshape(()),
            clus.reshape(()))
